# baseline (speedup 1.0000x reference)
import functools

import jax
import jax.numpy as jnp
import numpy as np
from jax import lax
from jax.experimental import pallas as pl
from jax.experimental.pallas import tpu as pltpu
from jax.experimental.pallas import tpu_sc as plsc

_N = 10000
_E = 320000
_D = 128
_G = 128
_L = 64
_NL = 3

_NC = 2
_NS = 16
_NW = _NC * _NS
_CH = 128
_KJ = 80
_EPT = _CH * _KJ
_EPAD = _NW * _EPT
_PADN = _EPAD - _E
_RPT = 632
_RPTL = 528
_NP = (_NS - 1) * _RPT + _RPTL
_NTRASH = _NP - _N

_BN = float(1.0 / np.sqrt(1.0 + 1e-5))

_mesh = plsc.VectorSubcoreMesh(
    core_axis_name="c", subcore_axis_name="s", num_cores=_NC, num_subcores=_NS
)


_NRB = 2
_NQB = 4


@functools.partial(
    pl.kernel,
    out_type=jax.ShapeDtypeStruct((_NC, _N, _D), jnp.float32),
    mesh=_mesh,
    scratch_types=[
        pltpu.VMEM((_NQB, 2, _CH), jnp.int32),
        pltpu.VMEM((_NRB, _CH, _D), jnp.float32),
        pltpu.VMEM_SHARED((_NP, _D), jnp.float32),
        pltpu.SemaphoreType.DMA,
        pltpu.SemaphoreType.DMA,
        pltpu.SemaphoreType.DMA,
        pltpu.SemaphoreType.DMA,
    ],
)
def _sc_agg(h_hbm, edge_hbm, z_hbm, out_hbm, idx, rows, agg_sh, isem, gsem,
            ssem0, ssem1):
    ssems = (ssem0, ssem1)
    c = lax.axis_index("c")
    s = lax.axis_index("s")
    wid = c * _NS + s
    ebase = wid * _EPT

    def _fetch(j, slot):
        pltpu.async_copy(edge_hbm.at[:, pl.ds(ebase + j * _CH, _CH)],
                         idx.at[slot], isem)

    _fetch(0, 0)
    _fetch(1, 1)

    zbase = s * _RPT

    @pl.when(s < _NS - 1)
    def _():
        pltpu.sync_copy(z_hbm, agg_sh.at[pl.ds(zbase, _RPT)])

    @pl.when(s == _NS - 1)
    def _():
        pltpu.sync_copy(z_hbm.at[pl.ds(0, _RPTL)],
                        agg_sh.at[pl.ds(zbase, _RPTL)])

    plsc.subcore_barrier()

    def _it(j, rb, q0, q1, wait_ssem, pref, wait_idx, gather_next):
        pltpu.make_async_copy(h_hbm.at[idx.at[q0, 0]], rows.at[rb],
                              gsem).wait()
        pltpu.async_copy(rows.at[rb], agg_sh.at[idx.at[q0, 1]], ssems[rb],
                         add=True)
        if wait_ssem:
            pltpu.make_async_copy(rows.at[1 - rb], agg_sh.at[idx.at[0, 1]],
                                  ssems[1 - rb]).wait()
        if pref:
            _fetch(j + 3, (q0 + 3) % _NQB)
        if wait_idx:
            pltpu.make_async_copy(edge_hbm.at[:, pl.ds(ebase, _CH)],
                                  idx.at[q1], isem).wait()
        if gather_next:
            pltpu.async_copy(h_hbm.at[idx.at[q1, 0]], rows.at[1 - rb], gsem)

    _fetch(2, 2)
    pltpu.make_async_copy(edge_hbm.at[:, pl.ds(ebase, _CH)], idx.at[0],
                          isem).wait()
    pltpu.async_copy(h_hbm.at[idx.at[0, 0]], rows.at[0], gsem)
    _it(0, 0, 0, 1, wait_ssem=False, pref=True, wait_idx=True,
        gather_next=True)

    def _grp(kk, carry):
        for b in range(4):
            j = kk * 4 + b + 1
            _it(j, (b + 1) % 2, (b + 1) % 4, (b + 2) % 4,
                wait_ssem=True, pref=True, wait_idx=True, gather_next=True)
        return carry

    ngrp = jnp.where(wid == _NW - 1, (20 - 4) // 4, (_KJ - 4) // 4)
    lax.fori_loop(0, ngrp, _grp, 0)

    jlast = ngrp * 4 + 1
    _it(jlast, 1, 1, 2, wait_ssem=True, pref=False, wait_idx=True,
        gather_next=True)
    _it(jlast + 1, 0, 2, 3, wait_ssem=True, pref=False, wait_idx=True,
        gather_next=True)
    _it(jlast + 2, 1, 3, 0, wait_ssem=True, pref=False, wait_idx=False,
        gather_next=False)
    pltpu.make_async_copy(rows.at[1], agg_sh.at[idx.at[0, 1]], ssems[1]).wait()
    plsc.subcore_barrier()

    obase = s * _RPT

    @pl.when(s < _NS - 1)
    def _():
        pltpu.sync_copy(
            agg_sh.at[pl.ds(obase, _RPT)], out_hbm.at[c].at[pl.ds(obase, _RPT)]
        )

    @pl.when(s == _NS - 1)
    def _():
        last = _N - (_NS - 1) * _RPT
        pltpu.sync_copy(
            agg_sh.at[pl.ds(obase, last)], out_hbm.at[c].at[pl.ds(obase, last)]
        )


_NB = 5
_BM = _N // _NB


def _mlp_body(h_ref, p_ref, w1_ref, b1_ref, a1_ref, g1_ref, be1_ref,
              w2_ref, b2_ref, a2_ref, o_ref):
    z = h_ref[...] + p_ref[0] + p_ref[1]
    t = lax.dot_general(z, w1_ref[...], (((1,), (1,)), ((), ())),
                        preferred_element_type=jnp.float32)
    t = t + b1_ref[...]
    t = jnp.where(t >= 0, t, a1_ref[...] * t)
    t = (t * _BN) * g1_ref[...] + be1_ref[...]
    u = lax.dot_general(t, w2_ref[...], (((1,), (1,)), ((), ())),
                        preferred_element_type=jnp.float32)
    u = u + b2_ref[...]
    o_ref[...] = jnp.where(u >= 0, u, a2_ref[...] * u)


_row_spec = pl.BlockSpec((_BM, _D), lambda i: (i, 0))
_par_spec = pl.BlockSpec((_NC, _BM, _D), lambda i: (0, i, 0))
_w_spec = pl.BlockSpec((_D, _D), lambda i: (0, 0))
_v_spec = pl.BlockSpec((1, _D), lambda i: (0, 0))

_mlp_call = pl.pallas_call(
    _mlp_body,
    grid=(_NB,),
    in_specs=[_row_spec, _par_spec, _w_spec, _v_spec, _v_spec, _v_spec,
              _v_spec, _w_spec, _v_spec, _v_spec],
    out_specs=_row_spec,
    out_shape=jax.ShapeDtypeStruct((_N, _D), jnp.float32),
    compiler_params=pltpu.CompilerParams(dimension_semantics=("arbitrary",)),
)


def _fin_body(h_ref, p_ref, w1_ref, b1_ref, a1_ref, g1_ref, be1_ref,
              w2_ref, b2_ref, a2_ref, bat_ref, gf_ref, bf_ref, fw_ref, fb_ref,
              o_ref, acc_ref):
    i = pl.program_id(0)
    z = h_ref[...] + p_ref[0] + p_ref[1]
    t = lax.dot_general(z, w1_ref[...], (((1,), (1,)), ((), ())),
                        preferred_element_type=jnp.float32)
    t = t + b1_ref[...]
    t = jnp.where(t >= 0, t, a1_ref[...] * t)
    t = (t * _BN) * g1_ref[...] + be1_ref[...]
    u = lax.dot_general(t, w2_ref[...], (((1,), (1,)), ((), ())),
                        preferred_element_type=jnp.float32)
    u = u + b2_ref[...]
    u = jnp.where(u >= 0, u, a2_ref[...] * u)

    b = bat_ref[0]
    gi = lax.broadcasted_iota(jnp.int32, (_G, _BM), 0)
    m = (b == gi).astype(jnp.float32)

    @pl.when(i == 0)
    def _():
        acc_ref[...] = jnp.zeros_like(acc_ref)

    acc_ref[...] += jnp.dot(m, u, preferred_element_type=jnp.float32)

    @pl.when(i == _NB - 1)
    def _():
        pooled = (acc_ref[...] * _BN) * gf_ref[...] + bf_ref[...]
        o_ref[...] = lax.dot_general(
            pooled, fw_ref[...], (((1,), (1,)), ((), ())),
            preferred_element_type=jnp.float32) + fb_ref[...]


_fin_call = pl.pallas_call(
    _fin_body,
    grid=(_NB,),
    in_specs=[_row_spec, _par_spec, _w_spec, _v_spec, _v_spec, _v_spec,
              _v_spec, _w_spec, _v_spec, _v_spec,
              pl.BlockSpec((1, 1, _BM), lambda i: (i, 0, 0)),
              pl.BlockSpec((1, _D), lambda i: (0, 0)),
              pl.BlockSpec((1, _D), lambda i: (0, 0)),
              pl.BlockSpec((_L, _D), lambda i: (0, 0)),
              pl.BlockSpec((1, _L), lambda i: (0, 0))],
    out_specs=pl.BlockSpec((_G, _L), lambda i: (0, 0)),
    out_shape=jax.ShapeDtypeStruct((_G, _L), jnp.float32),
    scratch_shapes=[pltpu.VMEM((_G, _D), jnp.float32)],
    compiler_params=pltpu.CompilerParams(dimension_semantics=("arbitrary",)),
)


def kernel(x, edge_index, batch, W1, b1, a1, g1, be1, W2, b2, a2, gf, bf, fcW, fcb):
    bat3 = batch.reshape(_NB, 1, _BM)
    b1r = b1.reshape(_NL, 1, _D)
    a1r = jnp.broadcast_to(a1[:, None, None], (_NL, 1, _D))
    g1r = g1.reshape(_NL, 1, _D)
    be1r = be1.reshape(_NL, 1, _D)
    b2r = b2.reshape(_NL, 1, _D)
    a2r = jnp.broadcast_to(a2[:, None, None], (_NL, 1, _D))
    gfr = gf.reshape(1, _D)
    bfr = bf.reshape(1, _D)
    fbr = fcb.reshape(1, _L)

    zer = jnp.zeros((_RPT, _D), jnp.float32)
    h = x
    for i in range(_NL - 1):
        p = _sc_agg(h, edge_index, zer)
        h = _mlp_call(h, p, W1[i], b1r[i], a1r[i], g1r[i], be1r[i],
                      W2[i], b2r[i], a2r[i])
    i = _NL - 1
    p = _sc_agg(h, edge_index, zer)
    return _fin_call(h, p, W1[i], b1r[i], a1r[i], g1r[i], be1r[i],
                     W2[i], b2r[i], a2r[i], bat3, gfr, bfr, fcW, fbr)


# speedup vs baseline: 1.0028x; 1.0028x over previous
import functools

import jax
import jax.numpy as jnp
import numpy as np
from jax import lax
from jax.experimental import pallas as pl
from jax.experimental.pallas import tpu as pltpu
from jax.experimental.pallas import tpu_sc as plsc

_N = 10000
_E = 320000
_D = 128
_G = 128
_L = 64
_NL = 3

_NC = 2
_NS = 16
_NW = _NC * _NS
_CH = 128
_KJ = 80
_EPT = _CH * _KJ
_RPT = 632
_RPTL = 528
_NP = (_NS - 1) * _RPT + _RPTL

_BN = float(1.0 / np.sqrt(1.0 + 1e-5))

_mesh = plsc.VectorSubcoreMesh(
    core_axis_name="c", subcore_axis_name="s", num_cores=_NC, num_subcores=_NS
)


_NRB = 2
_NQB = 4


@functools.partial(
    pl.kernel,
    out_type=jax.ShapeDtypeStruct((_NC, _N, _D), jnp.float32),
    mesh=_mesh,
    scratch_types=[
        pltpu.VMEM((_NQB, 2, _CH), jnp.int32),
        pltpu.VMEM((_NRB, _CH, _D), jnp.float32),
        pltpu.VMEM_SHARED((_NP, _D), jnp.float32),
        pltpu.SemaphoreType.DMA,
        pltpu.SemaphoreType.DMA,
        pltpu.SemaphoreType.DMA,
        pltpu.SemaphoreType.DMA,
    ],
)
def _sc_agg(h_hbm, edge_hbm, z_hbm, out_hbm, idx, rows, agg_sh, isem, gsem,
            ssem0, ssem1):
    ssems = (ssem0, ssem1)
    c = lax.axis_index("c")
    s = lax.axis_index("s")
    wid = c * _NS + s
    ebase = wid * _EPT

    def _fetch(j, slot):
        pltpu.async_copy(edge_hbm.at[:, pl.ds(ebase + j * _CH, _CH)],
                         idx.at[slot], isem)

    _fetch(0, 0)
    _fetch(1, 1)

    zbase = s * _RPT

    @pl.when(s < _NS - 1)
    def _():
        pltpu.sync_copy(z_hbm, agg_sh.at[pl.ds(zbase, _RPT)])

    @pl.when(s == _NS - 1)
    def _():
        pltpu.sync_copy(z_hbm.at[pl.ds(0, _RPTL)],
                        agg_sh.at[pl.ds(zbase, _RPTL)])

    plsc.subcore_barrier()

    def _it(j, rb, q0, q1, wait_ssem, pref, wait_idx, gather_next):
        pltpu.make_async_copy(h_hbm.at[idx.at[q0, 0]], rows.at[rb],
                              gsem).wait()
        pltpu.async_copy(rows.at[rb], agg_sh.at[idx.at[q0, 1]], ssems[rb],
                         add=True)
        if wait_ssem:
            pltpu.make_async_copy(rows.at[1 - rb], agg_sh.at[idx.at[0, 1]],
                                  ssems[1 - rb]).wait()
        if pref:
            _fetch(j + 3, (q0 + 3) % _NQB)
        if wait_idx:
            pltpu.make_async_copy(edge_hbm.at[:, pl.ds(ebase, _CH)],
                                  idx.at[q1], isem).wait()
        if gather_next:
            pltpu.async_copy(h_hbm.at[idx.at[q1, 0]], rows.at[1 - rb], gsem)

    _fetch(2, 2)
    pltpu.make_async_copy(edge_hbm.at[:, pl.ds(ebase, _CH)], idx.at[0],
                          isem).wait()
    pltpu.async_copy(h_hbm.at[idx.at[0, 0]], rows.at[0], gsem)
    _it(0, 0, 0, 1, wait_ssem=False, pref=True, wait_idx=True,
        gather_next=True)

    def _grp(kk, carry):
        for b in range(4):
            j = kk * 4 + b + 1
            _it(j, (b + 1) % 2, (b + 1) % 4, (b + 2) % 4,
                wait_ssem=True, pref=True, wait_idx=True, gather_next=True)
        return carry

    ngrp = jnp.where(wid == _NW - 1, (20 - 4) // 4, (_KJ - 4) // 4)
    lax.fori_loop(0, ngrp, _grp, 0)

    jlast = ngrp * 4 + 1
    _it(jlast, 1, 1, 2, wait_ssem=True, pref=False, wait_idx=True,
        gather_next=True)
    _it(jlast + 1, 0, 2, 3, wait_ssem=True, pref=False, wait_idx=True,
        gather_next=True)
    _it(jlast + 2, 1, 3, 0, wait_ssem=True, pref=False, wait_idx=False,
        gather_next=False)
    pltpu.make_async_copy(rows.at[1], agg_sh.at[idx.at[0, 1]], ssems[1]).wait()
    plsc.subcore_barrier()

    obase = s * _RPT

    @pl.when(s < _NS - 1)
    def _():
        pltpu.sync_copy(
            agg_sh.at[pl.ds(obase, _RPT)], out_hbm.at[c].at[pl.ds(obase, _RPT)]
        )

    @pl.when(s == _NS - 1)
    def _():
        last = _N - (_NS - 1) * _RPT
        pltpu.sync_copy(
            agg_sh.at[pl.ds(obase, last)], out_hbm.at[c].at[pl.ds(obase, last)]
        )


_NB = 5
_BM = _N // _NB


def _mlp_body(h_ref, p_ref, w1_ref, b1_ref, a1_ref, g1_ref, be1_ref,
              w2_ref, b2_ref, a2_ref, o_ref):
    z = h_ref[...] + p_ref[0] + p_ref[1]
    t = lax.dot_general(z, w1_ref[...], (((1,), (1,)), ((), ())),
                        preferred_element_type=jnp.float32)
    t = t + b1_ref[...]
    t = jnp.where(t >= 0, t, a1_ref[...] * t)
    t = (t * _BN) * g1_ref[...] + be1_ref[...]
    u = lax.dot_general(t, w2_ref[...], (((1,), (1,)), ((), ())),
                        preferred_element_type=jnp.float32)
    u = u + b2_ref[...]
    o_ref[...] = jnp.where(u >= 0, u, a2_ref[...] * u)


_row_spec = pl.BlockSpec((_BM, _D), lambda i: (i, 0))
_par_spec = pl.BlockSpec((_NC, _BM, _D), lambda i: (0, i, 0))
_w_spec = pl.BlockSpec((_D, _D), lambda i: (0, 0))
_v_spec = pl.BlockSpec((1, _D), lambda i: (0, 0))

_mlp_call = pl.pallas_call(
    _mlp_body,
    grid=(_NB,),
    in_specs=[_row_spec, _par_spec, _w_spec, _v_spec, _v_spec, _v_spec,
              _v_spec, _w_spec, _v_spec, _v_spec],
    out_specs=_row_spec,
    out_shape=jax.ShapeDtypeStruct((_N, _D), jnp.float32),
    compiler_params=pltpu.CompilerParams(dimension_semantics=("arbitrary",)),
)


def _fin_body(h_ref, p_ref, w1_ref, b1_ref, a1_ref, g1_ref, be1_ref,
              w2_ref, b2_ref, a2_ref, bat_ref, gf_ref, bf_ref, fw_ref, fb_ref,
              o_ref, acc_ref):
    i = pl.program_id(0)
    z = h_ref[...] + p_ref[0] + p_ref[1]
    t = lax.dot_general(z, w1_ref[...], (((1,), (1,)), ((), ())),
                        preferred_element_type=jnp.float32)
    t = t + b1_ref[...]
    t = jnp.where(t >= 0, t, a1_ref[...] * t)
    t = (t * _BN) * g1_ref[...] + be1_ref[...]
    u = lax.dot_general(t, w2_ref[...], (((1,), (1,)), ((), ())),
                        preferred_element_type=jnp.float32)
    u = u + b2_ref[...]
    u = jnp.where(u >= 0, u, a2_ref[...] * u)

    b = bat_ref[0]
    gi = lax.broadcasted_iota(jnp.int32, (_G, _BM), 0)
    m = (b == gi).astype(jnp.float32)

    @pl.when(i == 0)
    def _():
        acc_ref[...] = jnp.zeros_like(acc_ref)

    acc_ref[...] += jnp.dot(m, u, preferred_element_type=jnp.float32)

    @pl.when(i == _NB - 1)
    def _():
        pooled = (acc_ref[...] * _BN) * gf_ref[...] + bf_ref[...]
        o_ref[...] = lax.dot_general(
            pooled, fw_ref[...], (((1,), (1,)), ((), ())),
            preferred_element_type=jnp.float32) + fb_ref[...]


_fin_call = pl.pallas_call(
    _fin_body,
    grid=(_NB,),
    in_specs=[_row_spec, _par_spec, _w_spec, _v_spec, _v_spec, _v_spec,
              _v_spec, _w_spec, _v_spec, _v_spec,
              pl.BlockSpec((1, 1, _BM), lambda i: (i, 0, 0)),
              pl.BlockSpec((1, _D), lambda i: (0, 0)),
              pl.BlockSpec((1, _D), lambda i: (0, 0)),
              pl.BlockSpec((_L, _D), lambda i: (0, 0)),
              pl.BlockSpec((1, _L), lambda i: (0, 0))],
    out_specs=pl.BlockSpec((_G, _L), lambda i: (0, 0)),
    out_shape=jax.ShapeDtypeStruct((_G, _L), jnp.float32),
    scratch_shapes=[pltpu.VMEM((_G, _D), jnp.float32)],
    compiler_params=pltpu.CompilerParams(dimension_semantics=("arbitrary",)),
)


def kernel(x, edge_index, batch, W1, b1, a1, g1, be1, W2, b2, a2, gf, bf, fcW, fcb):
    bat3 = batch.reshape(_NB, 1, _BM)
    b1r = b1.reshape(_NL, 1, _D)
    a1r = jnp.broadcast_to(a1[:, None, None], (_NL, 1, _D))
    g1r = g1.reshape(_NL, 1, _D)
    be1r = be1.reshape(_NL, 1, _D)
    b2r = b2.reshape(_NL, 1, _D)
    a2r = jnp.broadcast_to(a2[:, None, None], (_NL, 1, _D))
    gfr = gf.reshape(1, _D)
    bfr = bf.reshape(1, _D)
    fbr = fcb.reshape(1, _L)

    zer = jnp.zeros((_RPT, _D), jnp.float32)
    h = x
    for i in range(_NL - 1):
        p = _sc_agg(h, edge_index, zer)
        h = _mlp_call(h, p, W1[i], b1r[i], a1r[i], g1r[i], be1r[i],
                      W2[i], b2r[i], a2r[i])
    i = _NL - 1
    p = _sc_agg(h, edge_index, zer)
    return _fin_call(h, p, W1[i], b1r[i], a1r[i], g1r[i], be1r[i],
                     W2[i], b2r[i], a2r[i], bat3, gfr, bfr, fcW, fbr)
